# baseline (speedup 1.0000x reference)
import jax
import jax.numpy as jnp
from jax import lax
from jax.experimental import pallas as pl
from jax.experimental.pallas import tpu as pltpu
from jax.experimental.pallas import tpu_sc as plsc

N_OUT = 10242
N_IN = 40962
FAN = 7
NEIGH = FAN - 1
D = 512
LANES = 16
NW = 32
C = 8
PW = 336
K = PW // C
PAD_N = NW * PW
IDX_PW = PW * NEIGH
CN = C * NEIGH
NSTREAM = 6
SW = CN // NSTREAM

_INV_FAN = 1.0 / FAN


def _pool_kernel(x_hbm, idx_hbm, out_hbm, idx_v, gbuf, sbuf, obuf,
                 gsem, ssem, osem):
    cid = lax.axis_index("c")
    sid = lax.axis_index("s")
    wid = sid * 2 + cid
    row0 = wid * PW

    pltpu.sync_copy(idx_hbm.at[pl.ds(wid * IDX_PW, IDX_PW)], idx_v)

    def gather_start(g, slot):
        for p in range(NSTREAM):
            idx_p = idx_v.at[pl.ds(g * CN + p * SW, SW)]
            pltpu.async_copy(x_hbm.at[idx_p], gbuf.at[slot, pl.ds(p * SW, SW)],
                             gsem.at[slot, p])
        pltpu.async_copy(x_hbm.at[pl.ds(row0 + g * C, C), :], sbuf.at[slot],
                         ssem.at[slot])

    def gather_wait(slot):
        for p in range(NSTREAM):
            pltpu.make_async_copy(
                x_hbm.at[idx_v.at[pl.ds(0, SW)]],
                gbuf.at[slot, pl.ds(p * SW, SW)], gsem.at[slot, p],
            ).wait()
        pltpu.make_async_copy(
            x_hbm.at[pl.ds(0, C), :], sbuf.at[slot], ssem.at[slot],
        ).wait()

    def out_start(g, slot):
        base = row0 + g * C
        nval = N_OUT - base

        @pl.when(nval >= C)
        def _full():
            pltpu.async_copy(
                obuf.at[slot], out_hbm.at[pl.ds(base, C), :], osem.at[slot]
            )

        @pl.when(jnp.logical_and(nval > 0, nval < C))
        def _part():
            pltpu.async_copy(
                obuf.at[slot, pl.ds(0, 2), :],
                out_hbm.at[pl.ds(base, 2), :],
                osem.at[slot],
            )

    def out_wait(g, slot):
        base = row0 + g * C
        nval = N_OUT - base

        @pl.when(nval >= C)
        def _full():
            pltpu.make_async_copy(
                obuf.at[slot], out_hbm.at[pl.ds(0, C), :], osem.at[slot]
            ).wait()

        @pl.when(jnp.logical_and(nval > 0, nval < C))
        def _part():
            pltpu.make_async_copy(
                obuf.at[slot, pl.ds(0, 2), :],
                out_hbm.at[pl.ds(0, 2), :],
                osem.at[slot],
            ).wait()

    def compute_chunk(slot):
        @pl.loop(0, C)
        def _rows(r):
            rbase = r * NEIGH

            def load_group(j):
                sl = pl.ds(j * LANES, LANES)
                v = [sbuf[slot, r, sl]]
                v += [gbuf[slot, rbase + k, sl] for k in range(NEIGH)]
                return v

            def reduce_store(j, v):
                acc = ((v[0] + v[1]) + (v[2] + v[3])) + ((v[4] + v[5]) + v[6])
                obuf[slot, r, pl.ds(j * LANES, LANES)] = acc * _INV_FAN

            prev = load_group(0)
            for j in range(1, D // LANES):
                cur = load_group(j)
                reduce_store(j - 1, prev)
                prev = cur
            reduce_store(D // LANES - 1, prev)

    gather_start(0, 0)
    gather_start(1, 1)

    @pl.loop(0, K, step=3)
    def _chunks(g):
        for b in range(3):
            gg = g + b

            @pl.when(gg + 2 < K)
            def _next():
                gather_start(gg + 2, (b + 2) % 3)

            gather_wait(b)

            @pl.when(gg >= 3)
            def _drain():
                out_wait(gg - 3, b)

            compute_chunk(b)
            out_start(gg, b)

    out_wait(K - 3, 0)
    out_wait(K - 2, 1)
    out_wait(K - 1, 2)


@jax.jit
def kernel(x, edge_index):
    src = edge_index[1].astype(jnp.int32)
    idx = src.reshape(N_OUT, FAN)[:, 1:]
    idx = jnp.concatenate(
        [idx, jnp.zeros((PAD_N - N_OUT, NEIGH), jnp.int32)], axis=0
    ).reshape(-1)

    mesh = plsc.VectorSubcoreMesh(core_axis_name="c", subcore_axis_name="s")
    run = pl.kernel(
        _pool_kernel,
        out_type=jax.ShapeDtypeStruct((N_OUT, D), jnp.float32),
        mesh=mesh,
        scratch_types=[
            pltpu.VMEM((IDX_PW,), jnp.int32),
            pltpu.VMEM((3, CN, D), jnp.float32),
            pltpu.VMEM((3, C, D), jnp.float32),
            pltpu.VMEM((3, C, D), jnp.float32),
            pltpu.SemaphoreType.DMA((3, NSTREAM)),
            pltpu.SemaphoreType.DMA((3,)),
            pltpu.SemaphoreType.DMA((3,)),
        ],
    )
    return run(x, idx)


# speedup vs baseline: 2.3624x; 2.3624x over previous
import jax
import jax.numpy as jnp
from jax import lax
from jax.experimental import pallas as pl
from jax.experimental.pallas import tpu as pltpu
from jax.experimental.pallas import tpu_sc as plsc

N_OUT = 10242
N_IN = 40962
FAN = 7
NEIGH = FAN - 1
D = 512
LANES = 16
NW = 32
C = 8
PW = 336
K = PW // C
PAD_N = NW * PW
IDX_PW = PW * NEIGH
CN = C * NEIGH
NSTREAM = 3
SW = CN // NSTREAM

_INV_FAN = 1.0 / FAN


def _pool_kernel(x_hbm, idx_hbm, out_hbm, idx_v, gbuf, sbuf, obuf,
                 gsem, ssem, osem):
    cid = lax.axis_index("c")
    sid = lax.axis_index("s")
    wid = sid * 2 + cid
    row0 = wid * PW

    pltpu.sync_copy(idx_hbm.at[pl.ds(wid * IDX_PW, IDX_PW)], idx_v)

    def gather_start(g, slot):
        for p in range(NSTREAM):
            idx_p = idx_v.at[pl.ds(g * CN + p * SW, SW)]
            pltpu.async_copy(x_hbm.at[idx_p], gbuf.at[slot, pl.ds(p * SW, SW)],
                             gsem.at[slot, p])
        pltpu.async_copy(x_hbm.at[pl.ds(row0 + g * C, C), :], sbuf.at[slot],
                         ssem.at[slot])

    def gather_wait(slot):
        for p in range(NSTREAM):
            pltpu.make_async_copy(
                x_hbm.at[idx_v.at[pl.ds(0, SW)]],
                gbuf.at[slot, pl.ds(p * SW, SW)], gsem.at[slot, p],
            ).wait()
        pltpu.make_async_copy(
            x_hbm.at[pl.ds(0, C), :], sbuf.at[slot], ssem.at[slot],
        ).wait()

    def out_start(g, slot):
        base = row0 + g * C
        nval = N_OUT - base

        @pl.when(nval >= C)
        def _full():
            pltpu.async_copy(
                obuf.at[slot], out_hbm.at[pl.ds(base, C), :], osem.at[slot]
            )

        @pl.when(jnp.logical_and(nval > 0, nval < C))
        def _part():
            pltpu.async_copy(
                obuf.at[slot, pl.ds(0, 2), :],
                out_hbm.at[pl.ds(base, 2), :],
                osem.at[slot],
            )

    def out_wait(g, slot):
        base = row0 + g * C
        nval = N_OUT - base

        @pl.when(nval >= C)
        def _full():
            pltpu.make_async_copy(
                obuf.at[slot], out_hbm.at[pl.ds(0, C), :], osem.at[slot]
            ).wait()

        @pl.when(jnp.logical_and(nval > 0, nval < C))
        def _part():
            pltpu.make_async_copy(
                obuf.at[slot, pl.ds(0, 2), :],
                out_hbm.at[pl.ds(0, 2), :],
                osem.at[slot],
            ).wait()

    def compute_chunk(slot):
        @pl.loop(0, C)
        def _rows(r):
            rbase = r * NEIGH

            def load_group(j):
                sl = pl.ds(j * LANES, LANES)
                v = [sbuf[slot, r, sl]]
                v += [gbuf[slot, rbase + k, sl] for k in range(NEIGH)]
                return v

            def reduce_store(j, v):
                acc = ((v[0] + v[1]) + (v[2] + v[3])) + ((v[4] + v[5]) + v[6])
                obuf[slot, r, pl.ds(j * LANES, LANES)] = acc * _INV_FAN

            prev = load_group(0)
            for j in range(1, D // LANES):
                cur = load_group(j)
                reduce_store(j - 1, prev)
                prev = cur
            reduce_store(D // LANES - 1, prev)

    gather_start(0, 0)
    gather_start(1, 1)

    @pl.loop(0, K, step=3)
    def _chunks(g):
        for b in range(3):
            gg = g + b

            @pl.when(gg + 2 < K)
            def _next():
                gather_start(gg + 2, (b + 2) % 3)

            gather_wait(b)

            @pl.when(gg >= 3)
            def _drain():
                out_wait(gg - 3, b)

            compute_chunk(b)
            out_start(gg, b)

    out_wait(K - 3, 0)
    out_wait(K - 2, 1)
    out_wait(K - 1, 2)


@jax.jit
def kernel(x, edge_index):
    src = edge_index[1].astype(jnp.int32)
    idx = src.reshape(N_OUT, FAN)[:, 1:]
    pad = (jnp.arange((PAD_N - N_OUT) * NEIGH, dtype=jnp.int32) * 197) % N_IN
    idx = jnp.concatenate([idx.reshape(-1), pad])

    mesh = plsc.VectorSubcoreMesh(core_axis_name="c", subcore_axis_name="s")
    run = pl.kernel(
        _pool_kernel,
        out_type=jax.ShapeDtypeStruct((N_OUT, D), jnp.float32),
        mesh=mesh,
        scratch_types=[
            pltpu.VMEM((IDX_PW,), jnp.int32),
            pltpu.VMEM((3, CN, D), jnp.float32),
            pltpu.VMEM((3, C, D), jnp.float32),
            pltpu.VMEM((3, C, D), jnp.float32),
            pltpu.SemaphoreType.DMA((3, NSTREAM)),
            pltpu.SemaphoreType.DMA((3,)),
            pltpu.SemaphoreType.DMA((3,)),
        ],
    )
    return run(x, idx)
